# baseline (speedup 1.0000x reference)
import jax
import jax.numpy as jnp
from jax import lax
from jax.experimental import pallas as pl
from jax.experimental.pallas import tpu as pltpu

T = 2048
HIDDEN = 768
NUM_EXPERTS = 8
INTER = 256
GROUP = 4
CHUNK = 1024


def _router_combine(xb, gate_w, bias_row, t):
    logits = lax.dot_general(xb, gate_w, (((1,), (1,)), ((), ())),
                             preferred_element_type=jnp.float32)
    scores = jax.nn.sigmoid(logits)
    scores_c = scores + bias_row

    def top2sum(s4):
        a, b, c, d = (s4[:, 0], s4[:, 1], s4[:, 2], s4[:, 3])
        return jnp.maximum(
            jnp.maximum(jnp.maximum(a + b, a + c), jnp.maximum(a + d, b + c)),
            jnp.maximum(b + d, c + d))

    g0 = top2sum(scores_c[:, 0:GROUP])
    g1 = top2sum(scores_c[:, GROUP:2 * GROUP])
    sel0 = (g0 >= g1).astype(jnp.float32)[:, None]
    lane = lax.broadcasted_iota(jnp.int32, (t, NUM_EXPERTS), 1)
    in_g0 = (lane < GROUP).astype(jnp.float32)
    maskf = sel0 * in_g0 + (1.0 - sel0) * (1.0 - in_g0)
    masked = scores_c * maskf - 1e9 * (1.0 - maskf)

    m = jnp.max(masked, axis=1, keepdims=True)
    eq = (masked == m).astype(jnp.float32)
    tri = (lax.broadcasted_iota(jnp.int32, (NUM_EXPERTS, NUM_EXPERTS), 0)
           < lax.broadcasted_iota(jnp.int32, (NUM_EXPERTS, NUM_EXPERTS), 1)
           ).astype(jnp.float32)
    prior = lax.dot_general(eq, tri, (((1,), (0,)), ((), ())),
                            preferred_element_type=jnp.float32)
    onehot = eq * (prior == 0.0).astype(jnp.float32)

    w = jnp.sum(onehot * scores, axis=1, keepdims=True)
    w = w / (w + 1e-20)
    return onehot * w


def _moe_body(x_ref, gate_w_ref, bias_ref, wgu_ref, wd_ref, sgu_ref, sd_ref,
              out_ref, comb_scr, x16_scr):
    e = pl.program_id(0)

    @pl.when(e == 0)
    def _router_and_shared():
        for s in range(T // CHUNK):
            sl = pl.ds(s * CHUNK, CHUNK)
            xb = x_ref[sl, :]
            comb_scr[sl, :] = _router_combine(
                xb, gate_w_ref[...], bias_ref[...], CHUNK)
            xb16 = xb.astype(jnp.bfloat16)
            x16_scr[sl, :] = xb16
            sgu = lax.dot_general(
                xb16, sgu_ref[...].astype(jnp.bfloat16),
                (((1,), (0,)), ((), ())),
                preferred_element_type=jnp.float32)
            sg = sgu[:, :INTER]
            su = sgu[:, INTER:]
            sh = (sg * jax.nn.sigmoid(sg) * su).astype(jnp.bfloat16)
            out_ref[sl, :] = lax.dot_general(
                sh, sd_ref[...].astype(jnp.bfloat16),
                (((1,), (0,)), ((), ())),
                preferred_element_type=jnp.float32)

    @pl.when(e > 0)
    def _expert():
        wgu16 = wgu_ref[0].astype(jnp.bfloat16)
        wd16 = wd_ref[0].astype(jnp.bfloat16)
        for s in range(T // CHUNK):
            sl = pl.ds(s * CHUNK, CHUNK)
            gu = lax.dot_general(x16_scr[sl, :], wgu16,
                                 (((1,), (0,)), ((), ())),
                                 preferred_element_type=jnp.float32)
            g = gu[:, :INTER]
            u = gu[:, INTER:]
            h = g * jax.nn.sigmoid(g) * u
            lane = lax.broadcasted_iota(jnp.int32, (CHUNK, NUM_EXPERTS), 1)
            cf = jnp.sum(
                comb_scr[sl, :] * (lane == (e - 1)).astype(jnp.float32),
                axis=1, keepdims=True)
            eo = lax.dot_general((h * cf).astype(jnp.bfloat16), wd16,
                                 (((1,), (0,)), ((), ())),
                                 preferred_element_type=jnp.float32)
            out_ref[sl, :] = out_ref[sl, :] + eo


def kernel(hidden_states, gate_w, correction_bias, w_gate_up, w_down,
           shared_gate_up, shared_down):
    bias_row = correction_bias.reshape(1, NUM_EXPERTS)
    grid = (1 + NUM_EXPERTS,)

    def wmap(e):
        return (jnp.maximum(e - 1, 0), 0, 0)

    return pl.pallas_call(
        _moe_body,
        grid=grid,
        in_specs=[
            pl.BlockSpec((T, HIDDEN), lambda e: (0, 0)),
            pl.BlockSpec((NUM_EXPERTS, HIDDEN), lambda e: (0, 0)),
            pl.BlockSpec((1, NUM_EXPERTS), lambda e: (0, 0)),
            pl.BlockSpec((1, HIDDEN, 2 * INTER), wmap),
            pl.BlockSpec((1, INTER, HIDDEN), wmap),
            pl.BlockSpec((HIDDEN, 2 * INTER), lambda e: (0, 0)),
            pl.BlockSpec((INTER, HIDDEN), lambda e: (0, 0)),
        ],
        out_specs=pl.BlockSpec((T, HIDDEN), lambda e: (0, 0)),
        out_shape=jax.ShapeDtypeStruct((T, HIDDEN), jnp.float32),
        scratch_shapes=[pltpu.VMEM((T, NUM_EXPERTS), jnp.float32),
                        pltpu.VMEM((T, HIDDEN), jnp.bfloat16)],
    )(hidden_states, gate_w, bias_row, w_gate_up, w_down,
      shared_gate_up, shared_down)


# speedup vs baseline: 1.0348x; 1.0348x over previous
import jax
import jax.numpy as jnp
from jax import lax
from jax.experimental import pallas as pl
from jax.experimental.pallas import tpu as pltpu

T = 2048
HIDDEN = 768
NUM_EXPERTS = 8
INTER = 256
GROUP = 4
CHUNK = 512


def _router_combine(xb, gate_w, bias_row, t):
    logits = lax.dot_general(xb, gate_w, (((1,), (1,)), ((), ())),
                             preferred_element_type=jnp.float32)
    scores = jax.nn.sigmoid(logits)
    scores_c = scores + bias_row

    def top2sum(s4):
        a, b, c, d = (s4[:, 0], s4[:, 1], s4[:, 2], s4[:, 3])
        return jnp.maximum(
            jnp.maximum(jnp.maximum(a + b, a + c), jnp.maximum(a + d, b + c)),
            jnp.maximum(b + d, c + d))

    g0 = top2sum(scores_c[:, 0:GROUP])
    g1 = top2sum(scores_c[:, GROUP:2 * GROUP])
    sel0 = (g0 >= g1).astype(jnp.float32)[:, None]
    lane = lax.broadcasted_iota(jnp.int32, (t, NUM_EXPERTS), 1)
    in_g0 = (lane < GROUP).astype(jnp.float32)
    maskf = sel0 * in_g0 + (1.0 - sel0) * (1.0 - in_g0)
    masked = scores_c * maskf - 1e9 * (1.0 - maskf)

    m = jnp.max(masked, axis=1, keepdims=True)
    eq = (masked == m).astype(jnp.float32)
    tri = (lax.broadcasted_iota(jnp.int32, (NUM_EXPERTS, NUM_EXPERTS), 0)
           < lax.broadcasted_iota(jnp.int32, (NUM_EXPERTS, NUM_EXPERTS), 1)
           ).astype(jnp.float32)
    prior = lax.dot_general(eq, tri, (((1,), (0,)), ((), ())),
                            preferred_element_type=jnp.float32)
    onehot = eq * (prior == 0.0).astype(jnp.float32)

    w = jnp.sum(onehot * scores, axis=1, keepdims=True)
    w = w / (w + 1e-20)
    return onehot * w


def _moe_body(x_ref, gate_w_ref, bias_ref, wgu_ref, wd_ref, sgu_ref, sd_ref,
              out_ref, comb_scr):
    e = pl.program_id(0)

    @pl.when(e == 0)
    def _router_and_shared():
        for s in range(T // CHUNK):
            sl = pl.ds(s * CHUNK, CHUNK)
            xb = x_ref[sl, :]
            comb_scr[sl, :] = _router_combine(
                xb, gate_w_ref[...], bias_ref[...], CHUNK)
            sgu = lax.dot_general(xb, sgu_ref[...], (((1,), (0,)), ((), ())),
                                  preferred_element_type=jnp.float32)
            sg = sgu[:, :INTER]
            su = sgu[:, INTER:]
            sh = sg * jax.nn.sigmoid(sg) * su
            out_ref[sl, :] = lax.dot_general(
                sh, sd_ref[...], (((1,), (0,)), ((), ())),
                preferred_element_type=jnp.float32)

    @pl.when(e > 0)
    def _expert():
        for s in range(T // CHUNK):
            sl = pl.ds(s * CHUNK, CHUNK)
            xb = x_ref[sl, :]
            gu = lax.dot_general(xb, wgu_ref[0], (((1,), (0,)), ((), ())),
                                 preferred_element_type=jnp.float32)
            g = gu[:, :INTER]
            u = gu[:, INTER:]
            h = g * jax.nn.sigmoid(g) * u
            lane = lax.broadcasted_iota(jnp.int32, (CHUNK, NUM_EXPERTS), 1)
            cf = jnp.sum(
                comb_scr[sl, :] * (lane == (e - 1)).astype(jnp.float32),
                axis=1, keepdims=True)
            eo = lax.dot_general(h * cf, wd_ref[0], (((1,), (0,)), ((), ())),
                                 preferred_element_type=jnp.float32)
            out_ref[sl, :] = out_ref[sl, :] + eo


def kernel(hidden_states, gate_w, correction_bias, w_gate_up, w_down,
           shared_gate_up, shared_down):
    bias_row = correction_bias.reshape(1, NUM_EXPERTS)
    grid = (1 + NUM_EXPERTS,)

    def wmap(e):
        return (jnp.maximum(e - 1, 0), 0, 0)

    return pl.pallas_call(
        _moe_body,
        grid=grid,
        in_specs=[
            pl.BlockSpec((T, HIDDEN), lambda e: (0, 0)),
            pl.BlockSpec((NUM_EXPERTS, HIDDEN), lambda e: (0, 0)),
            pl.BlockSpec((1, NUM_EXPERTS), lambda e: (0, 0)),
            pl.BlockSpec((1, HIDDEN, 2 * INTER), wmap),
            pl.BlockSpec((1, INTER, HIDDEN), wmap),
            pl.BlockSpec((HIDDEN, 2 * INTER), lambda e: (0, 0)),
            pl.BlockSpec((INTER, HIDDEN), lambda e: (0, 0)),
        ],
        out_specs=pl.BlockSpec((T, HIDDEN), lambda e: (0, 0)),
        out_shape=jax.ShapeDtypeStruct((T, HIDDEN), jnp.float32),
        scratch_shapes=[pltpu.VMEM((T, NUM_EXPERTS), jnp.float32)],
    )(hidden_states, gate_w, bias_row, w_gate_up, w_down,
      shared_gate_up, shared_down)
